# Initial kernel scaffold; baseline (speedup 1.0000x reference)
import jax
import jax.numpy as jnp
from jax.experimental import pallas as pl


def kernel(captions, table):
    raise NotImplementedError("write your pallas kernel here")


# speedup vs baseline: 4.6663x; 4.6663x over previous
import functools

import jax
import jax.numpy as jnp
from jax import lax
from jax.experimental import pallas as pl
from jax.experimental.pallas import tpu as pltpu
from jax.experimental.pallas import tpu_sc as plsc


def _make_sc_gather(V, D, B, n_workers):
    assert B % n_workers == 0
    b_per_w = B // n_workers
    C = 800
    assert b_per_w % C == 0
    n_chunks = b_per_w // C

    mesh = plsc.VectorSubcoreMesh(core_axis_name="c", subcore_axis_name="s")

    @functools.partial(
        pl.kernel,
        mesh=mesh,
        compiler_params=pltpu.CompilerParams(use_tc_tiling_on_sc=False),
        out_type=jax.ShapeDtypeStruct((B, D), jnp.float32),
        scratch_types=[
            pltpu.VMEM((b_per_w,), jnp.int32),
            pltpu.VMEM((C, D), jnp.float32),
            pltpu.VMEM((C, D), jnp.float32),
            pltpu.SemaphoreType.DMA,
            pltpu.SemaphoreType.DMA,
        ],
    )
    def gather_kernel(table_hbm, idx_hbm, out_hbm, idx_v, rows0, rows1, sem0, sem1):
        n_cores = lax.axis_size("c")
        wid = lax.axis_index("s") * n_cores + lax.axis_index("c")
        base = wid * b_per_w

        pltpu.sync_copy(idx_hbm.at[pl.ds(base, b_per_w)], idx_v)

        rows = (rows0, rows1)
        sems = (sem0, sem1)

        pltpu.async_copy(
            table_hbm.at[idx_v.at[pl.ds(0, C)]], rows[0], sems[0]
        )
        for c in range(n_chunks):
            cur = c % 2
            nxt = 1 - cur
            if c + 1 < n_chunks:
                pltpu.async_copy(
                    table_hbm.at[idx_v.at[pl.ds((c + 1) * C, C)]],
                    rows[nxt],
                    sems[nxt],
                )
            pltpu.make_async_copy(
                table_hbm.at[idx_v.at[pl.ds(c * C, C)]], rows[cur], sems[cur]
            ).wait()
            pltpu.sync_copy(rows[cur], out_hbm.at[pl.ds(base + c * C, C)])

    return gather_kernel


def kernel(captions, table):
    B, S = captions.shape
    V, D = table.shape
    flat_idx = captions.reshape(B * S).astype(jnp.int32)
    info = plsc.get_sparse_core_info()
    n_workers = info.num_cores * info.num_subcores
    out = _make_sc_gather(V, D, B * S, n_workers)(table, flat_idx)
    return out.reshape(B, S, D)
